# Initial kernel scaffold; baseline (speedup 1.0000x reference)
import jax
import jax.numpy as jnp
from jax.experimental import pallas as pl


def kernel(x, edge_index, W_enc, b_enc, Wm1, Wm2, bm, Wo1, Wo2, bo):
    raise NotImplementedError("write your pallas kernel here")


# speedup vs baseline: 6.3413x; 6.3413x over previous
import functools

import jax
import jax.numpy as jnp
from jax import lax
from jax.experimental import pallas as pl
from jax.experimental.pallas import tpu as pltpu
from jax.experimental.pallas import tpu_sc as plsc

N = 10000
E = 320000
D = 128
H = 128

ROWS = 2000
C = 128
NCHUNK = E // C
NW = 32
TRIPS_BASE = NCHUNK // NW
TRIPS_REM = NCHUNK - TRIPS_BASE * NW
NPAD = 10240
RPT = NPAD // 16


def _enc_body(x_ref, we_ref, be_ref, wm1_ref, wm2_ref, bm_ref,
              h_ref, a_ref, b_ref):
    h = jnp.maximum(
        jnp.dot(x_ref[...], we_ref[...], preferred_element_type=jnp.float32)
        + be_ref[...], 0.0)
    h_ref[...] = h
    a_ref[...] = jnp.dot(h, wm1_ref[...],
                         preferred_element_type=jnp.float32) + bm_ref[...]
    b_ref[...] = jnp.dot(h, wm2_ref[...], preferred_element_type=jnp.float32)


def _encode(x, W_enc, b_enc, Wm1, Wm2, bm):
    grid = (N // ROWS,)
    blk_rows = pl.BlockSpec((ROWS, D), lambda i: (i, 0))
    blk_w = pl.BlockSpec((D, H), lambda i: (0, 0))
    blk_b = pl.BlockSpec((1, H), lambda i: (0, 0))
    return pl.pallas_call(
        _enc_body,
        grid=grid,
        in_specs=[blk_rows, blk_w, blk_b, blk_w, blk_w, blk_b],
        out_specs=[blk_rows, blk_rows, blk_rows],
        out_shape=[jax.ShapeDtypeStruct((N, H), jnp.float32),
                   jax.ShapeDtypeStruct((NPAD, H), jnp.float32),
                   jax.ShapeDtypeStruct((NPAD, H), jnp.float32)],
    )(x, W_enc, b_enc, Wm1, Wm2, bm)


def _dec_body(h_ref, g0_ref, g1_ref, wo1_ref, wo2_ref, bo_ref, o_ref):
    agg = g0_ref[...] + g1_ref[...]
    o_ref[...] = jnp.maximum(
        jnp.dot(h_ref[...], wo1_ref[...], preferred_element_type=jnp.float32)
        + jnp.dot(agg, wo2_ref[...], preferred_element_type=jnp.float32)
        + bo_ref[...], 0.0)


def _decode(h, g0, g1, Wo1, Wo2, bo):
    grid = (N // ROWS,)
    blk_rows = pl.BlockSpec((ROWS, H), lambda i: (i, 0))
    blk_w = pl.BlockSpec((H, H), lambda i: (0, 0))
    blk_b = pl.BlockSpec((1, H), lambda i: (0, 0))
    return pl.pallas_call(
        _dec_body,
        grid=grid,
        in_specs=[blk_rows, blk_rows, blk_rows, blk_w, blk_w, blk_b],
        out_specs=blk_rows,
        out_shape=jax.ShapeDtypeStruct((N, H), jnp.float32),
    )(h, g0, g1, Wo1, Wo2, bo)


def _edge_body(src_hbm, dst_hbm, a_hbm, b_hbm, zeros_hbm, out_hbm,
               sidx, didx, arows, brows, agg, sem0, sem1):
    c = lax.axis_index("c")
    s = lax.axis_index("s")
    wid = s * 2 + c

    pltpu.sync_copy(zeros_hbm.at[pl.ds(s * RPT, RPT)],
                    agg.at[pl.ds(s * RPT, RPT)])
    plsc.subcore_barrier()

    trips = TRIPS_BASE + jnp.where(wid < TRIPS_REM, 1, 0)

    def chunk(t, carry):
        base = (wid + t * NW) * C
        pltpu.sync_copy(src_hbm.at[pl.ds(base, C)], sidx)
        pltpu.sync_copy(dst_hbm.at[pl.ds(base, C)], didx)
        cp0 = pltpu.async_copy(a_hbm.at[sidx], arows, sem0)
        cp1 = pltpu.async_copy(b_hbm.at[didx], brows, sem1)
        cp0.wait()
        cp1.wait()

        def erow(r, cc):
            for q in range(H // 16):
                sl = pl.ds(q * 16, 16)
                arows[r, sl] = jnp.maximum(arows[r, sl] + brows[r, sl], 0.0)
            return cc

        lax.fori_loop(0, C, erow, 0)
        pltpu.sync_copy(arows, agg.at[didx], add=True)
        return carry

    lax.fori_loop(0, trips, chunk, 0)
    plsc.subcore_barrier()
    pltpu.sync_copy(agg.at[pl.ds(s * RPT, RPT)],
                    out_hbm.at[c, pl.ds(s * RPT, RPT)])


def _edge(src, dst, A, B, zeros):
    mesh = plsc.VectorSubcoreMesh(core_axis_name="c", subcore_axis_name="s")
    fn = functools.partial(
        pl.kernel,
        out_type=jax.ShapeDtypeStruct((2, NPAD, H), jnp.float32),
        mesh=mesh,
        scratch_types=[
            pltpu.VMEM((C,), jnp.int32),
            pltpu.VMEM((C,), jnp.int32),
            pltpu.VMEM((C, H), jnp.float32),
            pltpu.VMEM((C, H), jnp.float32),
            pltpu.VMEM_SHARED((NPAD, H), jnp.float32),
            pltpu.SemaphoreType.DMA,
            pltpu.SemaphoreType.DMA,
        ],
    )(_edge_body)
    return fn(src, dst, A, B, zeros)


def kernel(x, edge_index, W_enc, b_enc, Wm1, Wm2, bm, Wo1, Wo2, bo):
    src = edge_index[0]
    dst = edge_index[1]
    h, A, B = _encode(x, W_enc, b_enc.reshape(1, H), Wm1, Wm2,
                      bm.reshape(1, H))
    parts = _edge(src, dst, A, B, jnp.zeros((NPAD, H), jnp.float32))
    return _decode(h, parts[0, :N], parts[1, :N], Wo1, Wo2, bo.reshape(1, H))


# speedup vs baseline: 10.9547x; 1.7275x over previous
import functools

import jax
import jax.numpy as jnp
from jax import lax
from jax.experimental import pallas as pl
from jax.experimental.pallas import tpu as pltpu
from jax.experimental.pallas import tpu_sc as plsc

N = 10000
E = 320000
D = 128
H = 128

ROWS = 2000
C = 48
NW = 32
EW = E // NW
TFULL = EW // C
TL = EW - TFULL * C
NPAD = 10240
RPT = NPAD // 16


def _enc_body(x_ref, we_ref, be_ref, wm1_ref, wm2_ref, bm_ref,
              h_ref, a_ref, b_ref):
    h = jnp.maximum(
        jnp.dot(x_ref[...], we_ref[...], preferred_element_type=jnp.float32)
        + be_ref[...], 0.0)
    h_ref[...] = h
    a_ref[...] = jnp.dot(h, wm1_ref[...],
                         preferred_element_type=jnp.float32) + bm_ref[...]
    b_ref[...] = jnp.dot(h, wm2_ref[...], preferred_element_type=jnp.float32)


def _encode(x, W_enc, b_enc, Wm1, Wm2, bm):
    grid = (N // ROWS,)
    blk_rows = pl.BlockSpec((ROWS, D), lambda i: (i, 0))
    blk_w = pl.BlockSpec((D, H), lambda i: (0, 0))
    blk_b = pl.BlockSpec((1, H), lambda i: (0, 0))
    return pl.pallas_call(
        _enc_body,
        grid=grid,
        in_specs=[blk_rows, blk_w, blk_b, blk_w, blk_w, blk_b],
        out_specs=[blk_rows, blk_rows, blk_rows],
        out_shape=[jax.ShapeDtypeStruct((N, H), jnp.float32),
                   jax.ShapeDtypeStruct((NPAD, H), jnp.float32),
                   jax.ShapeDtypeStruct((NPAD, H), jnp.float32)],
    )(x, W_enc, b_enc, Wm1, Wm2, bm)


def _dec_body(h_ref, g0_ref, g1_ref, wo1_ref, wo2_ref, bo_ref, o_ref):
    agg = g0_ref[...] + g1_ref[...]
    o_ref[...] = jnp.maximum(
        jnp.dot(h_ref[...], wo1_ref[...], preferred_element_type=jnp.float32)
        + jnp.dot(agg, wo2_ref[...], preferred_element_type=jnp.float32)
        + bo_ref[...], 0.0)


def _decode(h, g0, g1, Wo1, Wo2, bo):
    grid = (N // ROWS,)
    blk_rows = pl.BlockSpec((ROWS, H), lambda i: (i, 0))
    blk_w = pl.BlockSpec((H, H), lambda i: (0, 0))
    blk_b = pl.BlockSpec((1, H), lambda i: (0, 0))
    return pl.pallas_call(
        _dec_body,
        grid=grid,
        in_specs=[blk_rows, blk_rows, blk_rows, blk_w, blk_w, blk_b],
        out_specs=blk_rows,
        out_shape=jax.ShapeDtypeStruct((N, H), jnp.float32),
    )(h, g0, g1, Wo1, Wo2, bo)


def _edge_body(src_hbm, dst_hbm, a_hbm, b_hbm, zeros_hbm, out_hbm,
               sidx0, didx0, sd0, ga0, gb0,
               sidx1, didx1, sd1, ga1, gb1,
               sidx2, didx2, sd2, ga2, gb2,
               tsidx, tdidx, tsd, tga, tgb, agg,
               gsem0, gsem1, gsem2, ssem0, ssem1, ssem2,
               isem0, isem1, isem2, tsem):
    c = lax.axis_index("c")
    s = lax.axis_index("s")
    wid = s * 2 + c
    wbase = wid * EW

    bufs = ((sidx0, didx0, sd0, ga0, gb0, gsem0, ssem0, isem0),
            (sidx1, didx1, sd1, ga1, gb1, gsem1, ssem1, isem1),
            (sidx2, didx2, sd2, ga2, gb2, gsem2, ssem2, isem2))

    def fire_idx(t, b):
        sidx, didx = bufs[b][0], bufs[b][1]
        isem = bufs[b][7]
        off = pl.ds(wbase + t * C, C)
        pltpu.async_copy(src_hbm.at[off], sidx, isem)
        pltpu.async_copy(dst_hbm.at[off], didx, isem)

    def wait_idx(b):
        sidx, didx = bufs[b][0], bufs[b][1]
        isem = bufs[b][7]
        pltpu.make_async_copy(src_hbm.at[pl.ds(0, C)], sidx, isem).wait()
        pltpu.make_async_copy(dst_hbm.at[pl.ds(0, C)], didx, isem).wait()

    def fire_gather(b):
        sidx, didx, _, ga, gb, gsem = bufs[b][:6]
        pltpu.async_copy(a_hbm.at[sidx], ga, gsem)
        pltpu.async_copy(b_hbm.at[didx], gb, gsem)

    def wait_gather(b):
        sidx, didx, _, ga, gb, gsem = bufs[b][:6]
        pltpu.make_async_copy(a_hbm.at[sidx], ga, gsem).wait()
        pltpu.make_async_copy(b_hbm.at[didx], gb, gsem).wait()

    def wait_scatter(b):
        _, _, sd, ga, _, _, ssem, _ = bufs[b]
        pltpu.make_async_copy(ga, agg.at[sd], ssem).wait()

    def substep(t, b, fire_next_gather, fire_next_idx):
        sidx, didx, sd, ga, gb, gsem, ssem, isem = bufs[b]
        bn = (b + 2) % 3
        wait_gather(b)

        @plsc.parallel_loop(0, C, 1, unroll=4)
        def _(r):
            for q in range(H // 16):
                sl = pl.ds(q * 16, 16)
                ga[r, sl] = jnp.maximum(ga[r, sl] + gb[r, sl], 0.0)

        for q in range(C // 16):
            sl = pl.ds(q * 16, 16)
            sd[sl] = didx[sl]
        pltpu.async_copy(ga, agg.at[sd], ssem, add=True)
        wait_scatter(bn)
        if fire_next_gather:
            wait_idx(bn)
            fire_gather(bn)
        if fire_next_idx:
            fire_idx(t + 3, b)

    off0 = pl.ds(wbase, C)
    pltpu.sync_copy(src_hbm.at[off0], sidx0)
    pltpu.sync_copy(dst_hbm.at[off0], didx0)
    off1 = pl.ds(wbase + C, C)
    pltpu.sync_copy(src_hbm.at[off1], sidx1)
    pltpu.sync_copy(dst_hbm.at[off1], didx1)
    off2 = pl.ds(wbase + 2 * C, C)
    pltpu.sync_copy(src_hbm.at[off2], sidx2)
    pltpu.sync_copy(dst_hbm.at[off2], didx2)
    fire_gather(0)
    fire_gather(1)

    pltpu.sync_copy(zeros_hbm.at[pl.ds(s * RPT, RPT)],
                    agg.at[pl.ds(s * RPT, RPT)])
    plsc.subcore_barrier()

    wait_gather(0)

    @plsc.parallel_loop(0, C, 1, unroll=4)
    def _(r):
        for q in range(H // 16):
            sl = pl.ds(q * 16, 16)
            ga0[r, sl] = jnp.maximum(ga0[r, sl] + gb0[r, sl], 0.0)

    for q in range(C // 16):
        sl = pl.ds(q * 16, 16)
        sd0[sl] = didx0[sl]
    pltpu.async_copy(ga0, agg.at[sd0], ssem0, add=True)
    fire_gather(2)
    fire_idx(3, 0)

    substep(1, 1, True, False)
    fire_idx(4, 1)

    substep(2, 2, True, True)

    def three(i, carry):
        t = 3 * i + 3
        substep(t, 0, True, True)
        substep(t + 1, 1, True, True)
        substep(t + 2, 2, True, True)
        return carry

    lax.fori_loop(0, (TFULL - 7) // 3, three, 0)

    substep(TFULL - 4, (TFULL - 4) % 3, True, True)
    substep(TFULL - 3, (TFULL - 3) % 3, True, False)

    toff = pl.ds(wbase + TFULL * C, TL)
    pltpu.sync_copy(src_hbm.at[toff], tsidx)
    pltpu.sync_copy(dst_hbm.at[toff], tdidx)
    pltpu.async_copy(a_hbm.at[tsidx], tga, tsem)
    pltpu.async_copy(b_hbm.at[tdidx], tgb, tsem)

    substep(TFULL - 2, (TFULL - 2) % 3, False, False)
    substep(TFULL - 1, (TFULL - 1) % 3, False, False)

    pltpu.make_async_copy(a_hbm.at[tsidx], tga, tsem).wait()
    pltpu.make_async_copy(b_hbm.at[tdidx], tgb, tsem).wait()

    @plsc.parallel_loop(0, TL, 1, unroll=4)
    def _(r):
        for q in range(H // 16):
            sl = pl.ds(q * 16, 16)
            tga[r, sl] = jnp.maximum(tga[r, sl] + tgb[r, sl], 0.0)

    tsd[pl.ds(0, TL)] = tdidx[pl.ds(0, TL)]
    pltpu.sync_copy(tga, agg.at[tsd], add=True)

    wait_scatter((TFULL - 1) % 3)
    plsc.subcore_barrier()
    pltpu.sync_copy(agg.at[pl.ds(s * RPT, RPT)],
                    out_hbm.at[c, pl.ds(s * RPT, RPT)])


def _edge(src, dst, A, B, zeros):
    mesh = plsc.VectorSubcoreMesh(core_axis_name="c", subcore_axis_name="s")
    buf = [
        pltpu.VMEM((C,), jnp.int32),
        pltpu.VMEM((C,), jnp.int32),
        pltpu.VMEM((C,), jnp.int32),
        pltpu.VMEM((C, H), jnp.float32),
        pltpu.VMEM((C, H), jnp.float32),
    ]
    fn = functools.partial(
        pl.kernel,
        out_type=jax.ShapeDtypeStruct((2, NPAD, H), jnp.float32),
        mesh=mesh,
        scratch_types=[
            *buf, *buf, *buf,
            pltpu.VMEM((TL,), jnp.int32),
            pltpu.VMEM((TL,), jnp.int32),
            pltpu.VMEM((TL,), jnp.int32),
            pltpu.VMEM((TL, H), jnp.float32),
            pltpu.VMEM((TL, H), jnp.float32),
            pltpu.VMEM_SHARED((NPAD, H), jnp.float32),
            pltpu.SemaphoreType.DMA,
            pltpu.SemaphoreType.DMA,
            pltpu.SemaphoreType.DMA,
            pltpu.SemaphoreType.DMA,
            pltpu.SemaphoreType.DMA,
            pltpu.SemaphoreType.DMA,
            pltpu.SemaphoreType.DMA,
            pltpu.SemaphoreType.DMA,
            pltpu.SemaphoreType.DMA,
            pltpu.SemaphoreType.DMA,
        ],
    )(_edge_body)
    return fn(src, dst, A, B, zeros)


def kernel(x, edge_index, W_enc, b_enc, Wm1, Wm2, bm, Wo1, Wo2, bo):
    src = edge_index[0]
    dst = edge_index[1]
    h, A, B = _encode(x, W_enc, b_enc.reshape(1, H), Wm1, Wm2,
                      bm.reshape(1, H))
    parts = _edge(src, dst, A, B, jnp.zeros((NPAD, H), jnp.float32))
    return _decode(h, parts[0, :N], parts[1, :N], Wo1, Wo2, bo.reshape(1, H))
